# Initial kernel scaffold; baseline (speedup 1.0000x reference)
import jax
import jax.numpy as jnp
from jax.experimental import pallas as pl


def kernel(x, edge_index, attention, W_z, b_z, Wl_z, bl_z, W_r, b_r, Wl_r, bl_r, W_h, b_h, Wl_h, bl_h, W1, b1, W3, b3, W4, b4):
    raise NotImplementedError("write your pallas kernel here")


# speedup vs baseline: 212.5828x; 212.5828x over previous
import jax
import jax.numpy as jnp
from jax.experimental import pallas as pl

_B = 16
_N = 207
_F = 4
_OUT = 256
_T = 12
_E = 6624
_NP = 256
_EP = 7168
_TF = _T * _F
_R = _B * _NP

_HI = jax.lax.Precision.HIGHEST


def _bf(a):
    return a.astype(jnp.bfloat16)


def _main_kernel(src_ref, dstT_ref, xn_ref, att_ref,
                 wz_ref, bz_ref, wlz_ref, blz_ref,
                 wh_ref, bh_ref, wlh_ref, blh_ref,
                 w1_ref, b1_ref, w3_ref, b3_ref,
                 w4rep_ref, b4_ref, out_ref):
    f32 = jnp.float32

    src = src_ref[...]
    dstT = dstT_ref[...]
    oh_src = (src == jax.lax.broadcasted_iota(jnp.int32, (_EP, _NP), 1)
              ).astype(f32)
    oh_dstT = (dstT == jax.lax.broadcasted_iota(jnp.int32, (_NP, _EP), 0)
               ).astype(f32)
    A = jnp.dot(oh_dstT, oh_src, preferred_element_type=f32)

    deg = jnp.sum(A, axis=1, keepdims=True)
    dinv = jnp.where(deg > 0, 1.0 / jnp.sqrt(deg), 0.0)
    S = A * dinv * jnp.reshape(dinv, (1, _NP))

    xbf = _bf(xn_ref[...]).astype(f32)
    P = jnp.concatenate(
        [jnp.dot(S, xbf[b], precision=_HI, preferred_element_type=f32)
         for b in range(_B)], axis=0)

    wzb = _bf(wz_ref[...]).astype(f32)
    whb = _bf(wh_ref[...]).astype(f32)
    azb = _bf(wlz_ref[0:_OUT, :])
    ahb = _bf(wlh_ref[0:_OUT, :])
    w1b = _bf(w1_ref[...])
    w3b = _bf(w3_ref[...])

    att = att_ref[...]
    e = jnp.exp(att - jnp.max(att, axis=1, keepdims=True))
    probs = e / jnp.sum(e, axis=1, keepdims=True)

    hacc = jnp.zeros((_R, _OUT), f32)
    for t in range(_T):
        Pt = P[:, t * _F:(t + 1) * _F]
        gz = jnp.dot(Pt, wzb, precision=_HI,
                     preferred_element_type=f32) + bz_ref[...]
        gh = jnp.dot(Pt, whb, precision=_HI,
                     preferred_element_type=f32) + bh_ref[...]
        az = jnp.dot(_bf(gz), azb, preferred_element_type=f32) + blz_ref[...]
        ah = jnp.dot(_bf(gh), ahb, preferred_element_type=f32) + blh_ref[...]
        hn = (1.0 - jax.nn.sigmoid(az)) * jnp.tanh(ah)
        hacc = hacc + probs[:, t:t + 1] * hn

    h1 = jnp.dot(_bf(hacc), w1b, preferred_element_type=f32) + b1_ref[...]
    h3 = jnp.dot(_bf(h1), w3b, preferred_element_type=f32) + b3_ref[...]

    rows_b = jax.lax.broadcasted_iota(jnp.int32, (_B, _R), 0)
    cols_b = jax.lax.broadcasted_iota(jnp.int32, (_B, _R), 1) // _NP
    w4m = _bf(jnp.where(rows_b == cols_b,
                        jnp.broadcast_to(w4rep_ref[...], (_B, _R)), 0.0))
    outv = jnp.dot(w4m, _bf(h3), preferred_element_type=f32) + b4_ref[...]
    out_ref[...] = jnp.maximum(outv, 0.0)


def kernel(x, edge_index, attention, W_z, b_z, Wl_z, bl_z, W_r, b_r, Wl_r,
           bl_r, W_h, b_h, Wl_h, bl_h, W1, b1, W3, b3, W4, b4):
    f32 = jnp.float32
    loop = jnp.arange(_N, dtype=jnp.int32)
    pad = jnp.full((_EP - _E - _N,), _NP + 7, jnp.int32)
    src = jnp.concatenate([edge_index[0], loop, pad]).reshape(_EP, 1)
    dst = jnp.concatenate([edge_index[1], loop, pad]).reshape(1, _EP)

    xn = jnp.transpose(x, (0, 1, 3, 2)).reshape(_B, _N, _TF)
    xn = jnp.pad(xn, ((0, 0), (0, _NP - _N), (0, 0)))

    w4rep = jnp.tile(jnp.pad(W4[:, 0], (0, _NP - _N)), _B).reshape(1, _R)

    out = pl.pallas_call(
        _main_kernel,
        out_shape=jax.ShapeDtypeStruct((_B, 1), f32),
    )(src, dst, xn, attention.reshape(1, _T),
      W_z, b_z.reshape(1, _OUT), Wl_z, bl_z.reshape(1, _OUT),
      W_h, b_h.reshape(1, _OUT), Wl_h, bl_h.reshape(1, _OUT),
      W1, b1.reshape(1, 128), W3, b3.reshape(1, 1),
      w4rep, b4.reshape(1, 1))
    return out.reshape(_B)
